# baseline (speedup 1.0000x reference)
import jax
import jax.numpy as jnp
from jax.experimental import pallas as pl
from jax.experimental.pallas import tpu as pltpu

EMBED = 1024
NUM_HEADS = 16
HEAD_DIM = EMBED // NUM_HEADS
NUM_BLOCKS = 64
BS = 64
N_RAND = 3
SCALING = HEAD_DIM ** (-0.5)
NEG_INF = -1e30


def _proj_body(x_ref, w_ref, b_ref, o_ref):
    acc = jnp.dot(x_ref[...], w_ref[...], preferred_element_type=jnp.float32)
    o_ref[...] = acc + b_ref[...]


def _qkv_proj(x2d, w_all, b_all, block_m=512):
    m = x2d.shape[0]
    n = w_all.shape[1]
    k = x2d.shape[1]
    return pl.pallas_call(
        _proj_body,
        grid=(m // block_m,),
        in_specs=[
            pl.BlockSpec((block_m, k), lambda i: (i, 0)),
            pl.BlockSpec((k, n), lambda i: (0, 0)),
            pl.BlockSpec((1, n), lambda i: (0, 0)),
        ],
        out_specs=pl.BlockSpec((block_m, n), lambda i: (i, 0)),
        out_shape=jax.ShapeDtypeStruct((m, n), jnp.float32),
    )(x2d, w_all, b_all)


def _attn_body(rand_ref, q_ref, k_ref, v_ref, o_ref):
    b = pl.program_id(0)
    h = pl.program_id(1)
    base = (b * NUM_HEADS + h) * NUM_BLOCKS * N_RAND

    def blk(i, mask_cols):
        q = q_ref[0, pl.ds(i * BS, BS), 0, 0, :]
        if isinstance(i, int):
            ws = min(max(i - 1, 0), NUM_BLOCKS - 3) * BS
        else:
            ws = jnp.minimum(jnp.maximum(i - 1, 0), NUM_BLOCKS - 3) * BS
        r0 = rand_ref[base + i * N_RAND]
        r1 = rand_ref[base + i * N_RAND + 1]
        r2 = rand_ref[base + i * N_RAND + 2]

        def qk(rows, n):
            return jax.lax.dot_general(
                q, k_ref[0, pl.ds(rows, n), 0, 0, :],
                (((1,), (1,)), ((), ())),
                preferred_element_type=jnp.float32)

        scores = jnp.concatenate(
            [qk(ws, 3 * BS), qk(r0 * BS, BS), qk(r1 * BS, BS), qk(r2 * BS, BS)],
            axis=1)

        if mask_cols is not None:
            lo, hi = mask_cols
            col = jax.lax.broadcasted_iota(jnp.int32, scores.shape, 1)
            scores = jnp.where((col >= lo) & (col < hi), NEG_INF, scores)

        p = jax.nn.softmax(scores, axis=-1)

        ctx = jnp.dot(p[:, :3 * BS], v_ref[0, pl.ds(ws, 3 * BS), 0, 0, :],
                      preferred_element_type=jnp.float32)
        for c, r in ((3, r0), (4, r1), (5, r2)):
            ctx += jnp.dot(p[:, c * BS:(c + 1) * BS],
                           v_ref[0, pl.ds(r * BS, BS), 0, 0, :],
                           preferred_element_type=jnp.float32)
        o_ref[0, pl.ds(i * BS, BS), 0, 0, :] = ctx

    blk(0, (2 * BS, 3 * BS))
    jax.lax.fori_loop(1, NUM_BLOCKS - 1, lambda i, c: (blk(i, None), c)[1], 0,
                      unroll=4)
    blk(NUM_BLOCKS - 1, (0, BS))


def _block_attn(qkv, rand_attn, bsz, seqlen):
    def col_spec(group):
        return pl.BlockSpec(
            (1, seqlen, 1, 1, HEAD_DIM),
            lambda b, h, rand_ref, g=group: (b, 0, g * NUM_HEADS + h, 0, 0))

    grid_spec = pltpu.PrefetchScalarGridSpec(
        num_scalar_prefetch=1,
        grid=(bsz, NUM_HEADS),
        in_specs=[col_spec(0), col_spec(1), col_spec(2)],
        out_specs=pl.BlockSpec((1, seqlen, 1, 1, HEAD_DIM),
                               lambda b, h, rand_ref: (b, 0, h, 0, 0)),
    )
    return pl.pallas_call(
        _attn_body,
        grid_spec=grid_spec,
        out_shape=jax.ShapeDtypeStruct((bsz, seqlen, NUM_HEADS, 1, HEAD_DIM),
                                       jnp.float32),
        compiler_params=pltpu.CompilerParams(
            dimension_semantics=("arbitrary", "arbitrary")),
    )(rand_attn.reshape(-1), qkv, qkv, qkv)


def kernel(hidden_states, rand_attn, Wq, bq, Wk, bk, Wv, bv):
    bsz, seqlen, embed = hidden_states.shape

    w_all = jnp.concatenate([Wq.T * SCALING, Wk.T, Wv.T], axis=1)
    b_all = jnp.concatenate([bq * SCALING, bk, bv]).reshape(1, 3 * embed)

    x2d = hidden_states.reshape(bsz * seqlen, embed)
    qkv = _qkv_proj(x2d, w_all, b_all)
    qkv = qkv.reshape(bsz, seqlen, 3 * NUM_HEADS, 1, HEAD_DIM)

    ctx = _block_attn(qkv, rand_attn.astype(jnp.int32), bsz, seqlen)
    return ctx.reshape(bsz, seqlen, embed)


# speedup vs baseline: 1.3817x; 1.3817x over previous
import jax
import jax.numpy as jnp
from jax.experimental import pallas as pl
from jax.experimental.pallas import tpu as pltpu

EMBED = 1024
NUM_HEADS = 16
HEAD_DIM = EMBED // NUM_HEADS
NUM_BLOCKS = 64
BS = 64
N_RAND = 3
SCALING = HEAD_DIM ** (-0.5)
NEG_INF = -1e30


def _proj_body(x_ref, w_ref, b_ref, o_ref):
    acc = jnp.dot(x_ref[...], w_ref[...], preferred_element_type=jnp.float32)
    o_ref[...] = acc + b_ref[...]


def _qkv_proj(x2d, w_all, b_all, block_m=512):
    m = x2d.shape[0]
    n = w_all.shape[1]
    k = x2d.shape[1]
    return pl.pallas_call(
        _proj_body,
        grid=(m // block_m,),
        in_specs=[
            pl.BlockSpec((block_m, k), lambda i: (i, 0)),
            pl.BlockSpec((k, n), lambda i: (0, 0)),
            pl.BlockSpec((1, n), lambda i: (0, 0)),
        ],
        out_specs=pl.BlockSpec((block_m, n), lambda i: (i, 0)),
        out_shape=jax.ShapeDtypeStruct((m, n), jnp.float32),
    )(x2d, w_all, b_all)


LANES = 2 * HEAD_DIM
G = 4


def _attn_body(rand_ref, *refs):
    nr = 2 * G * N_RAND
    q_ref = refs[0]
    kc, khl, khr = refs[1], refs[2], refs[3]
    kr = refs[4:4 + nr]
    vc, vhl, vhr = refs[4 + nr], refs[5 + nr], refs[6 + nr]
    vr = refs[7 + nr:7 + 2 * nr]
    o_ref = refs[7 + 2 * nr]

    g = pl.program_id(2)
    first_grp = g == 0
    last_grp = g == (NUM_BLOCKS // G - 1)

    lane = jax.lax.broadcasted_iota(jnp.int32, (BS, LANES), 1)
    hmasks = ((lane < HEAD_DIM).astype(jnp.float32),
              (lane >= HEAD_DIM).astype(jnp.float32))

    for g0 in range(G):
        q_pair = q_ref[g0]
        kwin = [khl[0] if g0 == 0 else kc[g0 - 1],
                kc[g0],
                khr[0] if g0 == G - 1 else kc[g0 + 1]]
        vwin = [vhl[0] if g0 == 0 else vc[g0 - 1],
                vc[g0],
                vhr[0] if g0 == G - 1 else vc[g0 + 1]]

        ctxs = []
        for half in range(2):
            j = (half * G + g0) * N_RAND
            k6 = jnp.concatenate(
                kwin + [kr[j][0], kr[j + 1][0], kr[j + 2][0]],
                axis=0)
            scores = jax.lax.dot_general(
                q_pair * hmasks[half], k6, (((1,), (1,)), ((), ())),
                preferred_element_type=jnp.float32)

            if g0 == 0:
                col = jax.lax.broadcasted_iota(jnp.int32, scores.shape, 1)
                scores = jnp.where(first_grp & (col < BS), NEG_INF, scores)
            if g0 == G - 1:
                col = jax.lax.broadcasted_iota(jnp.int32, scores.shape, 1)
                scores = jnp.where(last_grp & (col >= 2 * BS) & (col < 3 * BS),
                                   NEG_INF, scores)

            p = jax.nn.softmax(scores, axis=-1)

            v6 = jnp.concatenate(
                vwin + [vr[j][0], vr[j + 1][0], vr[j + 2][0]],
                axis=0)
            ctxs.append(jnp.dot(p, v6, preferred_element_type=jnp.float32))

        o_ref[g0] = jnp.where(lane < HEAD_DIM, ctxs[0], ctxs[1])


def _block_attn(qkv, rand_attn, bsz, seqlen):
    n_pairs = NUM_HEADS // 2
    n_grps = NUM_BLOCKS // G

    def grp_spec(grp_off):
        return pl.BlockSpec(
            (None, G, BS, LANES),
            lambda b, a, g, rand_ref: (b, g, 0, grp_off + a))

    def halo_spec(grp_off, right):
        def idx(b, a, g, rand_ref):
            if right:
                return (b, jnp.minimum(g * G + G, NUM_BLOCKS - 1), 0,
                        grp_off + a)
            return (b, jnp.maximum(g * G - 1, 0), 0, grp_off + a)
        return pl.BlockSpec((None, 1, BS, LANES), idx)

    def rand_spec(grp_off, half, g0, r):
        def idx(b, a, g, rand_ref):
            head = 2 * a + half
            blk = g * G + g0
            flat = ((b * NUM_HEADS + head) * NUM_BLOCKS + blk) * N_RAND + r
            return (b, rand_ref[flat], 0, grp_off + a)
        return pl.BlockSpec((None, 1, BS, LANES), idx)

    def side(grp_off):
        return ([grp_spec(grp_off), halo_spec(grp_off, False),
                 halo_spec(grp_off, True)]
                + [rand_spec(grp_off, h, q, r)
                   for h in range(2) for q in range(G) for r in range(N_RAND)])

    in_specs = [grp_spec(0)] + side(n_pairs) + side(2 * n_pairs)

    grid_spec = pltpu.PrefetchScalarGridSpec(
        num_scalar_prefetch=1,
        grid=(bsz, n_pairs, n_grps),
        in_specs=in_specs,
        out_specs=pl.BlockSpec((None, G, BS, LANES),
                               lambda b, a, g, rand_ref: (b, g, 0, a)),
    )
    n_in = len(in_specs)
    return pl.pallas_call(
        _attn_body,
        grid_spec=grid_spec,
        out_shape=jax.ShapeDtypeStruct(
            (bsz, NUM_BLOCKS, BS, NUM_HEADS * HEAD_DIM), jnp.float32),
        compiler_params=pltpu.CompilerParams(
            dimension_semantics=("parallel", "parallel", "arbitrary")),
    )(rand_attn.reshape(-1), *([qkv] * n_in))


def kernel(hidden_states, rand_attn, Wq, bq, Wk, bk, Wv, bv):
    bsz, seqlen, embed = hidden_states.shape

    w_all = jnp.concatenate([Wq.T * SCALING, Wk.T, Wv.T], axis=1)
    b_all = jnp.concatenate([bq * SCALING, bk, bv]).reshape(1, 3 * embed)

    x2d = hidden_states.reshape(bsz * seqlen, embed)
    qkv = _qkv_proj(x2d, w_all, b_all)
    qkv = qkv.reshape(bsz, NUM_BLOCKS, BS, 3 * embed)

    ctx = _block_attn(qkv, rand_attn.astype(jnp.int32), bsz, seqlen)
    return ctx.reshape(bsz, seqlen, embed)
